# baseline (speedup 1.0000x reference)
import functools

import numpy as np

import jax
import jax.numpy as jnp
from jax import lax
from jax.experimental import pallas as pl
from jax.experimental.pallas import tpu as pltpu
from jax.experimental.pallas import tpu_sc as plsc

_LN2 = 0.6931471805599453

_NC = 2
_NS = 16
_NW = _NC * _NS
_L = 16


def _mlp_body(e_ref, w1_ref, b1_ref, w2_ref, b2_ref, o_ref):
    h = jnp.dot(e_ref[...], w1_ref[...], preferred_element_type=jnp.float32)
    h = h + b1_ref[...]
    sp = jnp.maximum(h, 0.0) + jnp.log1p(jnp.exp(-jnp.abs(h))) - _LN2
    o = jnp.dot(sp, w2_ref[...], preferred_element_type=jnp.float32)
    o_ref[...] = (o + b2_ref[...]).astype(o_ref.dtype)


def _edge_mlp(edges, W1, b1, W2, b2, block_e, blk0, n_blk, out_dtype):
    E, F = edges.shape
    U = W2.shape[1]
    return pl.pallas_call(
        _mlp_body,
        grid=(n_blk,),
        in_specs=[
            pl.BlockSpec((block_e, F), lambda i: (i + blk0, 0)),
            pl.BlockSpec((F, U), lambda i: (0, 0)),
            pl.BlockSpec((1, U), lambda i: (0, 0)),
            pl.BlockSpec((U, U), lambda i: (0, 0)),
            pl.BlockSpec((1, U), lambda i: (0, 0)),
        ],
        out_specs=pl.BlockSpec((block_e, U), lambda i: (i, 0)),
        out_shape=jax.ShapeDtypeStruct((n_blk * block_e, U), out_dtype),
    )(edges, W1, b1.reshape(1, U), W2, b2.reshape(1, U))


def _sc_gather_mul_scatter(x, nodes, src, dst, CH, NB, chunk0, dep=None):
    Ek, F = x.shape
    N = nodes.shape[0]
    n_chunks = Ek // CH
    base_w = n_chunks // _NW
    rem = n_chunks - base_w * _NW
    arows = ((N // _NS) + 7) // 8 * 8
    last_rows = N - (_NS - 1) * arows

    mesh = plsc.VectorSubcoreMesh(core_axis_name="c", subcore_axis_name="s")

    @functools.partial(
        pl.kernel,
        out_type=jax.ShapeDtypeStruct((_NC, N, F), jnp.float32),
        mesh=mesh,
        scratch_types=(
            [pltpu.VMEM((CH,), jnp.int32)] * NB
            + [pltpu.VMEM((CH,), jnp.int32)] * NB
            + [pltpu.VMEM((CH, F), x.dtype)] * NB
            + [pltpu.VMEM((CH, F), jnp.float32)] * NB
            + [pltpu.VMEM_SHARED((N, F), jnp.float32)]
            + [pltpu.SemaphoreType.DMA] * (5 * NB)
        ),
    )
    def sc_kernel(x_hbm, nodes_hbm, src_hbm, dst_hbm, zeros_hbm, dep_hbm,
                  out_hbm, *scr):
        src_v = scr[:NB]
        dst_v = scr[NB:2 * NB]
        x_v = scr[2 * NB:3 * NB]
        g_v = scr[3 * NB:4 * NB]
        acc_sh = scr[4 * NB]
        sems = scr[4 * NB + 1:]
        sem_sc = sems[:NB]
        sem_dt = sems[NB:2 * NB]
        sem_x = sems[2 * NB:3 * NB]
        sem_g = sems[3 * NB:4 * NB]
        sem_s = sems[4 * NB:5 * NB]

        c = lax.axis_index("c")
        s = lax.axis_index("s")
        wid = s * _NC + c
        first = wid * base_w + jnp.minimum(wid, rem)
        myn = base_w + jnp.where(wid < rem, 1, 0)

        @pl.when(s < _NS - 1)
        def _():
            pltpu.sync_copy(zeros_hbm.at[pl.ds(0, arows)],
                            acc_sh.at[pl.ds(s * arows, arows)])

        @pl.when(s == _NS - 1)
        def _():
            pltpu.sync_copy(zeros_hbm.at[pl.ds(0, last_rows)],
                            acc_sh.at[pl.ds((_NS - 1) * arows, last_rows)])

        plsc.subcore_barrier()

        def start_src(j, b):
            pltpu.async_copy(src_hbm.at[pl.ds((chunk0 + first + j) * CH, CH)],
                             src_v[b], sem_sc[b])

        def start_dst(j, b):
            pltpu.async_copy(dst_hbm.at[pl.ds((chunk0 + first + j) * CH, CH)],
                             dst_v[b], sem_dt[b])

        def start_fetch(j, b):
            pltpu.make_async_copy(src_hbm.at[pl.ds(0, CH)],
                                  src_v[b], sem_sc[b]).wait()
            pltpu.async_copy(x_hbm.at[pl.ds((first + j) * CH, CH)],
                             x_v[b], sem_x[b])
            pltpu.async_copy(nodes_hbm.at[src_v[b]], g_v[b], sem_g[b])

        def wait_fetch(b):
            pltpu.make_async_copy(x_hbm.at[pl.ds(0, CH)], x_v[b], sem_x[b]).wait()
            pltpu.make_async_copy(x_hbm.at[pl.ds(0, CH)], g_v[b], sem_g[b]).wait()

        def wait_dst(b):
            pltpu.make_async_copy(src_hbm.at[pl.ds(0, CH)],
                                  dst_v[b], sem_dt[b]).wait()

        def wait_scatter(b):
            pltpu.make_async_copy(x_hbm.at[pl.ds(0, CH)], g_v[b], sem_s[b]).wait()

        for b in range(NB):
            start_src(b, b)
            start_dst(b, b)
        for b in range(NB):
            start_fetch(b, b)

        @pl.loop(0, myn, step=NB)
        def _trip(i):
            for b in range(NB):
                j = i + b

                @pl.when(j < myn)
                def _():
                    wait_fetch(b)

                    @pl.when(j + NB < myn)
                    def _():
                        start_src(j + NB, b)

                    @pl.loop(0, CH)
                    def _mul(r):
                        for q in range(F // _L):
                            sl = pl.ds(q * _L, _L)
                            g_v[b][r, sl] = g_v[b][r, sl] * x_v[b][r, sl]

                    wait_dst(b)
                    pltpu.async_copy(g_v[b], acc_sh.at[dst_v[b]],
                                     sem_s[b], add=True)

            for b in range(NB):
                j = i + b

                @pl.when(j < myn)
                def _():
                    wait_scatter(b)

                    @pl.when(j + NB < myn)
                    def _():
                        start_dst(j + NB, b)
                        start_fetch(j + NB, b)

        plsc.subcore_barrier()

        @pl.when(s < _NS - 1)
        def _():
            pltpu.sync_copy(acc_sh.at[pl.ds(s * arows, arows)],
                            out_hbm.at[c].at[pl.ds(s * arows, arows)])

        @pl.when(s == _NS - 1)
        def _():
            pltpu.sync_copy(acc_sh.at[pl.ds((_NS - 1) * arows, last_rows)],
                            out_hbm.at[c].at[pl.ds((_NS - 1) * arows, last_rows)])

    zeros = jnp.zeros((arows, F), jnp.float32)
    if dep is None:
        dep = jnp.zeros((8, 128), jnp.float32)
    return sc_kernel(x, nodes, src, dst, zeros, dep)


def _psum_body(*refs):
    o_ref = refs[-1]
    acc = None
    for p_ref in refs[:-1]:
        term = p_ref[0] + p_ref[1]
        acc = term if acc is None else acc + term
    o_ref[...] = acc


def _partials_sum(partial_list, block_n):
    _, N, F = partial_list[0].shape
    return pl.pallas_call(
        _psum_body,
        grid=(N // block_n,),
        in_specs=[pl.BlockSpec((_NC, block_n, F), lambda i: (0, i, 0))
                  for _ in partial_list],
        out_specs=pl.BlockSpec((block_n, F), lambda i: (i, 0)),
        out_shape=jax.ShapeDtypeStruct((N, F), jnp.float32),
    )(*partial_list)


def kernel(nodes, edges, edge_index, W1, b1, W2, b2):
    E = edges.shape[0]
    U = W2.shape[1]
    CH = 64
    BLK = 2000
    slabs = (48000, 136000, 136000)
    src = edge_index[1]
    dst = edge_index[0]
    partial_list = []
    e0 = 0
    for Ek in slabs:
        xk = _edge_mlp(edges, W1, b1, W2, b2, block_e=BLK, blk0=e0 // BLK,
                       n_blk=Ek // BLK, out_dtype=jnp.float32)
        partial_list.append(
            _sc_gather_mul_scatter(xk, nodes, src, dst, CH=CH, NB=3,
                                   chunk0=e0 // CH,
                                   dep=partial_list[-1] if partial_list else None))
        e0 += Ek
    return _partials_sum(partial_list, block_n=2000)


# speedup vs baseline: 1.0216x; 1.0216x over previous
import functools

import numpy as np

import jax
import jax.numpy as jnp
from jax import lax
from jax.experimental import pallas as pl
from jax.experimental.pallas import tpu as pltpu
from jax.experimental.pallas import tpu_sc as plsc

_LN2 = 0.6931471805599453

_NC = 2
_NS = 16
_NW = _NC * _NS
_L = 16


def _mlp_body(e_ref, w1_ref, b1_ref, w2_ref, b2_ref, o_ref):
    h = jnp.dot(e_ref[...], w1_ref[...], preferred_element_type=jnp.float32)
    h = h + b1_ref[...]
    sp = jnp.maximum(h, 0.0) + jnp.log1p(jnp.exp(-jnp.abs(h))) - _LN2
    o = jnp.dot(sp, w2_ref[...], preferred_element_type=jnp.float32)
    o_ref[...] = (o + b2_ref[...]).astype(o_ref.dtype)


def _edge_mlp(edges, W1, b1, W2, b2, block_e, blk0, n_blk, out_dtype):
    E, F = edges.shape
    U = W2.shape[1]
    return pl.pallas_call(
        _mlp_body,
        grid=(n_blk,),
        in_specs=[
            pl.BlockSpec((block_e, F), lambda i: (i + blk0, 0)),
            pl.BlockSpec((F, U), lambda i: (0, 0)),
            pl.BlockSpec((1, U), lambda i: (0, 0)),
            pl.BlockSpec((U, U), lambda i: (0, 0)),
            pl.BlockSpec((1, U), lambda i: (0, 0)),
        ],
        out_specs=pl.BlockSpec((block_e, U), lambda i: (i, 0)),
        out_shape=jax.ShapeDtypeStruct((n_blk * block_e, U), out_dtype),
    )(edges, W1, b1.reshape(1, U), W2, b2.reshape(1, U))


def _sc_gather_mul_scatter(x, nodes, src, dst, CH, NB, chunk0, dep=None):
    Ek, F = x.shape
    N = nodes.shape[0]
    n_chunks = Ek // CH
    base_w = n_chunks // _NW
    rem = n_chunks - base_w * _NW
    arows = ((N // _NS) + 7) // 8 * 8
    last_rows = N - (_NS - 1) * arows

    mesh = plsc.VectorSubcoreMesh(core_axis_name="c", subcore_axis_name="s")

    @functools.partial(
        pl.kernel,
        out_type=jax.ShapeDtypeStruct((_NC, N, F), jnp.float32),
        mesh=mesh,
        scratch_types=(
            [pltpu.VMEM((CH,), jnp.int32)] * NB
            + [pltpu.VMEM((CH,), jnp.int32)] * NB
            + [pltpu.VMEM((CH, F), x.dtype)] * NB
            + [pltpu.VMEM((CH, F), jnp.float32)] * NB
            + [pltpu.VMEM_SHARED((N, F), jnp.float32)]
            + [pltpu.SemaphoreType.DMA] * (5 * NB)
        ),
    )
    def sc_kernel(x_hbm, nodes_hbm, src_hbm, dst_hbm, zeros_hbm, dep_hbm,
                  out_hbm, *scr):
        src_v = scr[:NB]
        dst_v = scr[NB:2 * NB]
        x_v = scr[2 * NB:3 * NB]
        g_v = scr[3 * NB:4 * NB]
        acc_sh = scr[4 * NB]
        sems = scr[4 * NB + 1:]
        sem_sc = sems[:NB]
        sem_dt = sems[NB:2 * NB]
        sem_x = sems[2 * NB:3 * NB]
        sem_g = sems[3 * NB:4 * NB]
        sem_s = sems[4 * NB:5 * NB]

        c = lax.axis_index("c")
        s = lax.axis_index("s")
        wid = s * _NC + c
        first = wid * base_w + jnp.minimum(wid, rem)
        myn = base_w + jnp.where(wid < rem, 1, 0)

        @pl.when(s < _NS - 1)
        def _():
            pltpu.sync_copy(zeros_hbm.at[pl.ds(0, arows)],
                            acc_sh.at[pl.ds(s * arows, arows)])

        @pl.when(s == _NS - 1)
        def _():
            pltpu.sync_copy(zeros_hbm.at[pl.ds(0, last_rows)],
                            acc_sh.at[pl.ds((_NS - 1) * arows, last_rows)])

        plsc.subcore_barrier()

        def start_src(j, b):
            pltpu.async_copy(src_hbm.at[pl.ds((chunk0 + first + j) * CH, CH)],
                             src_v[b], sem_sc[b])

        def start_dst(j, b):
            pltpu.async_copy(dst_hbm.at[pl.ds((chunk0 + first + j) * CH, CH)],
                             dst_v[b], sem_dt[b])

        def start_fetch(j, b):
            pltpu.make_async_copy(src_hbm.at[pl.ds(0, CH)],
                                  src_v[b], sem_sc[b]).wait()
            pltpu.async_copy(x_hbm.at[pl.ds((first + j) * CH, CH)],
                             x_v[b], sem_x[b])
            pltpu.async_copy(nodes_hbm.at[src_v[b]], g_v[b], sem_g[b])

        def wait_fetch(b):
            pltpu.make_async_copy(x_hbm.at[pl.ds(0, CH)], x_v[b], sem_x[b]).wait()
            pltpu.make_async_copy(x_hbm.at[pl.ds(0, CH)], g_v[b], sem_g[b]).wait()

        def wait_dst(b):
            pltpu.make_async_copy(src_hbm.at[pl.ds(0, CH)],
                                  dst_v[b], sem_dt[b]).wait()

        def wait_scatter(b):
            pltpu.make_async_copy(x_hbm.at[pl.ds(0, CH)], g_v[b], sem_s[b]).wait()

        for b in range(NB):
            start_src(b, b)
            start_dst(b, b)
        for b in range(NB):
            start_fetch(b, b)

        @pl.loop(0, myn, step=NB)
        def _trip(i):
            for b in range(NB):
                j = i + b

                @pl.when(j < myn)
                def _():
                    wait_fetch(b)

                    @pl.when(j + NB < myn)
                    def _():
                        start_src(j + NB, b)

                    @pl.loop(0, CH)
                    def _mul(r):
                        for q in range(F // _L):
                            sl = pl.ds(q * _L, _L)
                            g_v[b][r, sl] = g_v[b][r, sl] * x_v[b][r, sl]

                    wait_dst(b)
                    pltpu.async_copy(g_v[b], acc_sh.at[dst_v[b]],
                                     sem_s[b], add=True)

            for b in range(NB):
                j = i + b

                @pl.when(j < myn)
                def _():
                    wait_scatter(b)

                    @pl.when(j + NB < myn)
                    def _():
                        start_dst(j + NB, b)
                        start_fetch(j + NB, b)

        plsc.subcore_barrier()

        @pl.when(s < _NS - 1)
        def _():
            pltpu.sync_copy(acc_sh.at[pl.ds(s * arows, arows)],
                            out_hbm.at[c].at[pl.ds(s * arows, arows)])

        @pl.when(s == _NS - 1)
        def _():
            pltpu.sync_copy(acc_sh.at[pl.ds((_NS - 1) * arows, last_rows)],
                            out_hbm.at[c].at[pl.ds((_NS - 1) * arows, last_rows)])

    zeros = jnp.zeros((arows, F), jnp.float32)
    if dep is None:
        dep = jnp.zeros((8, 128), jnp.float32)
    return sc_kernel(x, nodes, src, dst, zeros, dep)


def _psum_body(*refs):
    o_ref = refs[-1]
    acc = None
    for p_ref in refs[:-1]:
        term = p_ref[0] + p_ref[1]
        acc = term if acc is None else acc + term
    o_ref[...] = acc


def _partials_sum(partial_list, block_n):
    _, N, F = partial_list[0].shape
    return pl.pallas_call(
        _psum_body,
        grid=(N // block_n,),
        in_specs=[pl.BlockSpec((_NC, block_n, F), lambda i: (0, i, 0))
                  for _ in partial_list],
        out_specs=pl.BlockSpec((block_n, F), lambda i: (i, 0)),
        out_shape=jax.ShapeDtypeStruct((N, F), jnp.float32),
    )(*partial_list)


def kernel(nodes, edges, edge_index, W1, b1, W2, b2):
    E = edges.shape[0]
    U = W2.shape[1]
    CH = 64
    BLK = 2000
    slabs = (64000, 128000, 128000)
    src = edge_index[1]
    dst = edge_index[0]
    partial_list = []
    e0 = 0
    for Ek in slabs:
        xk = _edge_mlp(edges, W1, b1, W2, b2, block_e=BLK, blk0=e0 // BLK,
                       n_blk=Ek // BLK, out_dtype=jnp.float32)
        partial_list.append(
            _sc_gather_mul_scatter(xk, nodes, src, dst, CH=CH, NB=3,
                                   chunk0=e0 // CH,
                                   dep=partial_list[-1] if partial_list else None))
        e0 += Ek
    return _partials_sum(partial_list, block_n=2000)
